# baseline (speedup 1.0000x reference)
import functools

import jax
import jax.numpy as jnp
from jax import lax
from jax.experimental import pallas as pl
from jax.experimental.pallas import tpu as pltpu
from jax.experimental.pallas import tpu_sc as plsc

_NC = 2
_NS = 16
_NW = _NC * _NS
_L = 16

_ROWS = 2 * 4096
_C = 4096
_ROWS_PER_W = _ROWS // _NW
_R = 4
_CH = _ROWS_PER_W // _R
_HALF = _CH // 2


def _body(x_hbm, perm_hbm, out_hbm, perm_v, in0, in1, ob0, ob1,
          si0, si1, so0, so1):
    cid = lax.axis_index("c")
    sid = lax.axis_index("s")
    wid = sid * _NC + cid
    base = wid * _ROWS_PER_W

    pltpu.sync_copy(perm_hbm, perm_v)

    def in_copy(ch, buf, sem):
        return pltpu.make_async_copy(
            x_hbm.at[pl.ds((base + ch * _R) * _C, _R * _C)], buf, sem)

    def out_copy(ch, buf, sem):
        return pltpu.make_async_copy(
            buf, out_hbm.at[pl.ds((base + ch * _R) * _C, _R * _C)], sem)

    def compute(inbuf, outbuf):
        def jstep(j, carry):
            idx = perm_v[pl.ds(j * _L, _L)]
            for r in range(_R):
                v = plsc.load_gather(inbuf, [idx + r * _C])
                outbuf[pl.ds(r * _C + j * _L, _L)] = v
            return carry
        lax.fori_loop(0, _C // _L, jstep, 0, unroll=2)

    in_copy(0, in0, si0).start()

    def pair(i, carry):
        ch0 = 2 * i
        in_copy(ch0, in0, si0).wait()
        in_copy(ch0 + 1, in1, si1).start()

        @pl.when(i > 0)
        def _():
            out_copy(ch0 - 2, ob0, so0).wait()

        compute(in0, ob0)
        out_copy(ch0, ob0, so0).start()

        in_copy(ch0 + 1, in1, si1).wait()

        @pl.when(i < _HALF - 1)
        def _():
            in_copy(ch0 + 2, in0, si0).start()

        @pl.when(i > 0)
        def _():
            out_copy(ch0 - 1, ob1, so1).wait()

        compute(in1, ob1)
        out_copy(ch0 + 1, ob1, so1).start()
        return carry

    lax.fori_loop(0, _HALF, pair, 0)
    out_copy(_CH - 2, ob0, so0).wait()
    out_copy(_CH - 1, ob1, so1).wait()


@jax.jit
def _shuffle(x2d, perm):
    mesh = plsc.VectorSubcoreMesh(core_axis_name="c", subcore_axis_name="s")
    f = functools.partial(
        pl.kernel,
        mesh=mesh,
        compiler_params=pltpu.CompilerParams(needs_layout_passes=False),
        out_type=jax.ShapeDtypeStruct((_ROWS * _C,), jnp.float32),
        scratch_types=[
            pltpu.VMEM((_C,), jnp.int32),
            pltpu.VMEM((_R * _C,), jnp.float32),
            pltpu.VMEM((_R * _C,), jnp.float32),
            pltpu.VMEM((_R * _C,), jnp.float32),
            pltpu.VMEM((_R * _C,), jnp.float32),
            pltpu.SemaphoreType.DMA,
            pltpu.SemaphoreType.DMA,
            pltpu.SemaphoreType.DMA,
            pltpu.SemaphoreType.DMA,
        ],
    )(_body)
    return f(x2d, perm)


def kernel(x):
    C = x.shape[-1]
    perm = jax.random.permutation(jax.random.key(42), C).astype(jnp.int32)
    out = _shuffle(x.reshape(-1), perm)
    return out.reshape(x.shape)


# speedup vs baseline: 1.7127x; 1.7127x over previous
import functools

import jax
import jax.numpy as jnp
from jax import lax
from jax.experimental import pallas as pl
from jax.experimental.pallas import tpu as pltpu
from jax.experimental.pallas import tpu_sc as plsc

_NC = 2
_NS = 16
_NW = _NC * _NS
_L = 16

_ROWS = 2 * 4096
_C = 4096
_ROWS_PER_W = _ROWS // _NW
_R = 4
_CH = _ROWS_PER_W // _R
_HALF = _CH // 2


def _body(x_hbm, perm_hbm, out_hbm, perm_v, in0, in1, ob0, ob1,
          si0, si1, so0, so1):
    cid = lax.axis_index("c")
    sid = lax.axis_index("s")
    wid = sid * _NC + cid
    base = wid * _ROWS_PER_W

    pltpu.sync_copy(perm_hbm, perm_v)

    def in_copy(ch, buf, sem):
        return pltpu.make_async_copy(
            x_hbm.at[pl.ds((base + ch * _R) * _C, _R * _C)], buf, sem)

    def out_copy(ch, buf, sem):
        return pltpu.make_async_copy(
            buf, out_hbm.at[pl.ds((base + ch * _R) * _C, _R * _C)], sem)

    def compute(inbuf, outbuf):
        @plsc.parallel_loop(0, _C // _L, unroll=4)
        def _jstep(j):
            idx = perm_v[pl.ds(j * _L, _L)]
            for r in range(_R):
                v = plsc.load_gather(inbuf, [idx + r * _C])
                outbuf[pl.ds(r * _C + j * _L, _L)] = v

    in_copy(0, in0, si0).start()

    def pair(i, carry):
        ch0 = 2 * i
        in_copy(ch0, in0, si0).wait()
        in_copy(ch0 + 1, in1, si1).start()

        @pl.when(i > 0)
        def _():
            out_copy(ch0 - 2, ob0, so0).wait()

        compute(in0, ob0)
        out_copy(ch0, ob0, so0).start()

        in_copy(ch0 + 1, in1, si1).wait()

        @pl.when(i < _HALF - 1)
        def _():
            in_copy(ch0 + 2, in0, si0).start()

        @pl.when(i > 0)
        def _():
            out_copy(ch0 - 1, ob1, so1).wait()

        compute(in1, ob1)
        out_copy(ch0 + 1, ob1, so1).start()
        return carry

    lax.fori_loop(0, _HALF, pair, 0)
    out_copy(_CH - 2, ob0, so0).wait()
    out_copy(_CH - 1, ob1, so1).wait()


@jax.jit
def _shuffle(x2d, perm):
    mesh = plsc.VectorSubcoreMesh(core_axis_name="c", subcore_axis_name="s")
    f = functools.partial(
        pl.kernel,
        mesh=mesh,
        compiler_params=pltpu.CompilerParams(needs_layout_passes=False),
        out_type=jax.ShapeDtypeStruct((_ROWS * _C,), jnp.float32),
        scratch_types=[
            pltpu.VMEM((_C,), jnp.int32),
            pltpu.VMEM((_R * _C,), jnp.float32),
            pltpu.VMEM((_R * _C,), jnp.float32),
            pltpu.VMEM((_R * _C,), jnp.float32),
            pltpu.VMEM((_R * _C,), jnp.float32),
            pltpu.SemaphoreType.DMA,
            pltpu.SemaphoreType.DMA,
            pltpu.SemaphoreType.DMA,
            pltpu.SemaphoreType.DMA,
        ],
    )(_body)
    return f(x2d, perm)


def kernel(x):
    C = x.shape[-1]
    perm = jax.random.permutation(jax.random.key(42), C).astype(jnp.int32)
    out = _shuffle(x.reshape(-1), perm)
    return out.reshape(x.shape)
